# baseline (speedup 1.0000x reference)
import dataclasses
import functools

import jax
import jax.numpy as jnp
from jax import lax
from jax.experimental import pallas as pl
from jax.experimental.pallas import tpu as pltpu
from jax.experimental.pallas import tpu_sc as plsc

NC = 2
NS = 16
NW = NC * NS


def _sc_compiler_params():
    cp = pltpu.CompilerParams()
    if "needs_layout_passes" in pltpu.CompilerParams.__dataclass_fields__:
        cp = dataclasses.replace(cp, needs_layout_passes=False)
    return cp


def _sc_gather(table, src2d, e_pad, ch):
    d = table.shape[1]
    mesh = plsc.VectorSubcoreMesh(core_axis_name="c", subcore_axis_name="s")

    @functools.partial(
        pl.kernel,
        mesh=mesh,
        out_type=jax.ShapeDtypeStruct((e_pad, d), table.dtype),
        scratch_types=[
            pltpu.VMEM((ch, 128), jnp.int32),
            pltpu.VMEM((128, d), table.dtype),
            pltpu.SemaphoreType.DMA,
        ],
    )
    def gk(table_hbm, src_hbm, out_hbm, idx_v, b0, s0):
        wid = lax.axis_index("s") * NC + lax.axis_index("c")
        pltpu.sync_copy(src_hbm.at[pl.ds(wid * ch, ch)], idx_v)

        @pl.loop(0, ch)
        def _(j):
            pltpu.async_copy(table_hbm.at[idx_v.at[j]], b0, s0).wait()
            pltpu.sync_copy(b0, out_hbm.at[pl.ds(wid * ch * 128 + j * 128, 128)])

    return gk(table, src2d)


def _sc_scatter_add(msg, dst1d, zeros_nk, n_pad, ch):
    k = msg.shape[1]
    half = n_pad // 2
    zrows = half // 8
    mesh = plsc.VectorSubcoreMesh(core_axis_name="c", subcore_axis_name="s")

    @functools.partial(
        pl.kernel,
        mesh=mesh,
        out_type=jax.ShapeDtypeStruct((NW, 2, zrows, 128), msg.dtype),
        compiler_params=_sc_compiler_params(),
        scratch_types=[
            pltpu.VMEM((zrows, 128), jnp.float32),
            pltpu.VMEM((128,), jnp.int32),
            pltpu.VMEM((128, k), jnp.float32),
            pltpu.SemaphoreType.DMA,
        ],
    )
    def sk(msg_hbm, dst_hbm, zeros_hbm, out_hbm, acc, idx1, mbuf, sem):
        c = lax.axis_index("c")
        s = lax.axis_index("s")
        wid = s * NC + c
        col = lax.iota(jnp.int32, 16)
        for p in range(2):
            pltpu.sync_copy(zeros_hbm, acc)

            @pl.loop(0, ch)
            def _(j):
                base = wid * ch * 128 + j * 128
                pltpu.sync_copy(dst_hbm.at[pl.ds(base, 128)], idx1)
                pltpu.sync_copy(msg_hbm.at[pl.ds(base, 128)], mbuf)

                @pl.loop(0, 8)
                def _(g):
                    rel = idx1[pl.ds(g * 16, 16)] - p * half
                    for l in range(16):
                        r = rel.at[jnp.full((16,), l, jnp.int32)].get(
                            mode="promise_in_bounds")
                        m = (r >= 0) & (r < half)
                        row = lax.shift_right_arithmetic(r, 3)
                        cp = lax.shift_left(r & 7, 4) + col
                        vals = mbuf[g * 16 + l, :]
                        plsc.addupdate_scatter(acc, [row, cp], vals, mask=m)

            pltpu.sync_copy(acc, out_hbm.at[wid, p])

    return sk(msg, dst1d, zeros_nk)


def _msg_matmul(hsrc, ef, w4, be):
    e_pad, d_in = hsrc.shape
    ed = ef.shape[1]
    k = w4.shape[1] // ed
    dsel = jnp.repeat(jnp.eye(ed, dtype=jnp.float32), k, axis=1)
    rsel = jnp.tile(jnp.eye(k, dtype=jnp.float32), (ed, 1))

    def body(h_ref, ef_ref, w_ref, s_ref, r_ref, o_ref):
        p = jnp.dot(h_ref[...], w_ref[...], preferred_element_type=jnp.float32)
        eft = jnp.dot(ef_ref[...], s_ref[...], preferred_element_type=jnp.float32)
        o_ref[...] = jnp.dot(p * eft, r_ref[...],
                             preferred_element_type=jnp.float32)

    return pl.pallas_call(
        body,
        grid=(e_pad // be,),
        in_specs=[
            pl.BlockSpec((be, d_in), lambda i: (i, 0)),
            pl.BlockSpec((be, ed), lambda i: (i, 0)),
            pl.BlockSpec((d_in, ed * k), lambda i: (0, 0)),
            pl.BlockSpec((ed, ed * k), lambda i: (0, 0)),
            pl.BlockSpec((ed * k, k), lambda i: (0, 0)),
        ],
        out_specs=pl.BlockSpec((be, k), lambda i: (i, 0)),
        out_shape=jax.ShapeDtypeStruct((e_pad, k), jnp.float32),
    )(hsrc, ef, w4, dsel, rsel)


def _reduce_partials(pp):
    nw, r, w = pp.shape

    def body(p_ref, o_ref):
        i = pl.program_id(0)

        @pl.when(i == 0)
        def _():
            o_ref[...] = p_ref[0]

        @pl.when(i > 0)
        def _():
            o_ref[...] += p_ref[0]

    return pl.pallas_call(
        body,
        grid=(nw,),
        in_specs=[pl.BlockSpec((1, r, w), lambda i: (i, 0, 0))],
        out_specs=pl.BlockSpec((r, w), lambda i: (0, 0)),
        out_shape=jax.ShapeDtypeStruct((r, w), jnp.float32),
    )(pp)


def _bn_relu(x, g, b, eps=1e-5):
    mean = jnp.mean(x, axis=0, keepdims=True)
    xc = x - mean
    var = jnp.mean(xc * xc, axis=0, keepdims=True)
    return jnp.maximum(g * xc / jnp.sqrt(var + eps) + b, 0.0)


def _tail(hn2d, h_self, w_self, w_neigh, gs, bs, gn, bn):
    n, k = h_self.shape[0], w_self.shape[1]

    def body(pp, hs, ws, wn, gsr, bsr, gnr, bnr, o):
        xs = jnp.dot(hs[...], ws[...], preferred_element_type=jnp.float32)
        zs = _bn_relu(xs, gsr[...], bsr[...])
        hn = pp[...]
        xn = jnp.dot(hn, wn[...], preferred_element_type=jnp.float32)
        zn = _bn_relu(xn, gnr[...], bnr[...])
        z = jnp.maximum(zs + zn, 0.0)
        nrm = jnp.sqrt(jnp.sum(z * z, axis=1, keepdims=True))
        nrm = jnp.where(nrm == 0.0, 1.0, nrm)
        o[...] = z / nrm

    return pl.pallas_call(
        body,
        out_shape=jax.ShapeDtypeStruct((n, k), jnp.float32),
    )(hn2d, h_self, w_self, w_neigh, gs, bs, gn, bn)


def kernel(h_neigh, h_self, edge_index, edge_features, W_edge, W_self, W_neigh,
           gamma_self, beta_self, gamma_neigh, beta_neigh):
    n, d_in = h_neigh.shape
    e = edge_index.shape[1]
    ed = edge_features.shape[1]
    k = W_self.shape[1]

    ch = -(-e // (NW * 128))
    e_pad = NW * ch * 128
    pad = e_pad - e
    src = jnp.concatenate([edge_index[0], jnp.zeros((pad,), jnp.int32)])
    dst = jnp.concatenate([edge_index[1], jnp.zeros((pad,), jnp.int32)])
    ef = jnp.concatenate([edge_features,
                          jnp.zeros((pad, ed), edge_features.dtype)])
    src2d = src.reshape(e_pad // 128, 128)
    dst2d = dst.reshape(e_pad // 128, 128)
    w4 = W_edge.reshape(ed, d_in, k).transpose(1, 0, 2).reshape(d_in, ed * k)
    hsrc = _sc_gather(h_neigh, src2d, e_pad, ch)
    msg = _msg_matmul(hsrc, ef, w4, 2048)
    n_pad = -(-n // 16) * 16
    zeros_nk = jnp.zeros((n_pad // 16, 128), jnp.float32)
    partials = _sc_scatter_add(msg, dst, zeros_nk, n_pad, ch)
    hn_pk = _reduce_partials(partials.reshape(NW, n_pad // 8, 128))
    hn2d = hn_pk.reshape(n_pad, k)[:n]
    return _tail(hn2d, h_self, W_self, W_neigh,
                 gamma_self.reshape(1, k), beta_self.reshape(1, k),
                 gamma_neigh.reshape(1, k), beta_neigh.reshape(1, k))


# speedup vs baseline: 1.0167x; 1.0167x over previous
import dataclasses
import functools

import jax
import jax.numpy as jnp
from jax import lax
from jax.experimental import pallas as pl
from jax.experimental.pallas import tpu as pltpu
from jax.experimental.pallas import tpu_sc as plsc

NC = 2
NS = 16
NW = NC * NS


def _sc_compiler_params():
    cp = pltpu.CompilerParams()
    if "needs_layout_passes" in pltpu.CompilerParams.__dataclass_fields__:
        cp = dataclasses.replace(cp, needs_layout_passes=False)
    return cp


def _sc_gather(table, src2d, e_pad, ch):
    d = table.shape[1]
    mesh = plsc.VectorSubcoreMesh(core_axis_name="c", subcore_axis_name="s")

    @functools.partial(
        pl.kernel,
        mesh=mesh,
        out_type=jax.ShapeDtypeStruct((e_pad, d), table.dtype),
        scratch_types=[
            pltpu.VMEM((ch, 128), jnp.int32),
            pltpu.VMEM((128, d), table.dtype),
            pltpu.VMEM((128, d), table.dtype),
            pltpu.VMEM((128, d), table.dtype),
            pltpu.VMEM((128, d), table.dtype),
            pltpu.SemaphoreType.DMA,
            pltpu.SemaphoreType.DMA,
            pltpu.SemaphoreType.DMA,
            pltpu.SemaphoreType.DMA,
        ],
    )
    def gk(table_hbm, src_hbm, out_hbm, idx_v, b0, b1, b2, b3, s0, s1, s2, s3):
        wid = lax.axis_index("s") * NC + lax.axis_index("c")
        pltpu.sync_copy(src_hbm.at[pl.ds(wid * ch, ch)], idx_v)

        @pl.loop(0, ch, step=4)
        def _(j):
            base = wid * ch * 128 + j * 128
            cps = [pltpu.async_copy(table_hbm.at[idx_v.at[j + q]], b, s)
                   for q, (b, s) in enumerate(((b0, s0), (b1, s1),
                                               (b2, s2), (b3, s3)))]
            for q, (b, cp) in enumerate(zip((b0, b1, b2, b3), cps)):
                cp.wait()
                pltpu.sync_copy(b, out_hbm.at[pl.ds(base + q * 128, 128)])

    return gk(table, src2d)


def _sc_scatter_add(msg, dst1d, zeros_nk, n_pad, ch):
    k = msg.shape[1]
    half = n_pad // 2
    zrows = half // 8
    mesh = plsc.VectorSubcoreMesh(core_axis_name="c", subcore_axis_name="s")

    @functools.partial(
        pl.kernel,
        mesh=mesh,
        out_type=jax.ShapeDtypeStruct((NW, 2, zrows, 128), msg.dtype),
        compiler_params=_sc_compiler_params(),
        scratch_types=[
            pltpu.VMEM((zrows, 128), jnp.float32),
            pltpu.VMEM((128,), jnp.int32),
            pltpu.VMEM((128, k), jnp.float32),
            pltpu.SemaphoreType.DMA,
        ],
    )
    def sk(msg_hbm, dst_hbm, zeros_hbm, out_hbm, acc, idx1, mbuf, sem):
        c = lax.axis_index("c")
        s = lax.axis_index("s")
        wid = s * NC + c
        col = lax.iota(jnp.int32, 16)
        for p in range(2):
            pltpu.sync_copy(zeros_hbm, acc)

            @pl.loop(0, ch)
            def _(j):
                base = wid * ch * 128 + j * 128
                pltpu.sync_copy(dst_hbm.at[pl.ds(base, 128)], idx1)
                pltpu.sync_copy(msg_hbm.at[pl.ds(base, 128)], mbuf)

                @pl.loop(0, 8)
                def _(g):
                    rel = idx1[pl.ds(g * 16, 16)] - p * half
                    for l in range(16):
                        r = rel.at[jnp.full((16,), l, jnp.int32)].get(
                            mode="promise_in_bounds")
                        m = (r >= 0) & (r < half)
                        row = lax.shift_right_arithmetic(r, 3)
                        cp = lax.shift_left(r & 7, 4) + col
                        vals = mbuf[g * 16 + l, :]
                        plsc.addupdate_scatter(acc, [row, cp], vals, mask=m)

            pltpu.sync_copy(acc, out_hbm.at[wid, p])

    return sk(msg, dst1d, zeros_nk)


def _msg_matmul(hsrc, ef, w4, be):
    e_pad, d_in = hsrc.shape
    ed = ef.shape[1]
    k = w4.shape[1] // ed
    dsel = jnp.repeat(jnp.eye(ed, dtype=jnp.float32), k, axis=1)
    rsel = jnp.tile(jnp.eye(k, dtype=jnp.float32), (ed, 1))

    def body(h_ref, ef_ref, w_ref, s_ref, r_ref, o_ref):
        p = jnp.dot(h_ref[...], w_ref[...], preferred_element_type=jnp.float32)
        eft = jnp.dot(ef_ref[...], s_ref[...], preferred_element_type=jnp.float32)
        o_ref[...] = jnp.dot(p * eft, r_ref[...],
                             preferred_element_type=jnp.float32)

    return pl.pallas_call(
        body,
        grid=(e_pad // be,),
        in_specs=[
            pl.BlockSpec((be, d_in), lambda i: (i, 0)),
            pl.BlockSpec((be, ed), lambda i: (i, 0)),
            pl.BlockSpec((d_in, ed * k), lambda i: (0, 0)),
            pl.BlockSpec((ed, ed * k), lambda i: (0, 0)),
            pl.BlockSpec((ed * k, k), lambda i: (0, 0)),
        ],
        out_specs=pl.BlockSpec((be, k), lambda i: (i, 0)),
        out_shape=jax.ShapeDtypeStruct((e_pad, k), jnp.float32),
    )(hsrc, ef, w4, dsel, rsel)


def _reduce_partials(pp):
    nw, r, w = pp.shape

    def body(p_ref, o_ref):
        i = pl.program_id(0)

        @pl.when(i == 0)
        def _():
            o_ref[...] = p_ref[0]

        @pl.when(i > 0)
        def _():
            o_ref[...] += p_ref[0]

    return pl.pallas_call(
        body,
        grid=(nw,),
        in_specs=[pl.BlockSpec((1, r, w), lambda i: (i, 0, 0))],
        out_specs=pl.BlockSpec((r, w), lambda i: (0, 0)),
        out_shape=jax.ShapeDtypeStruct((r, w), jnp.float32),
    )(pp)


def _bn_relu(x, g, b, eps=1e-5):
    mean = jnp.mean(x, axis=0, keepdims=True)
    xc = x - mean
    var = jnp.mean(xc * xc, axis=0, keepdims=True)
    return jnp.maximum(g * xc / jnp.sqrt(var + eps) + b, 0.0)


def _tail(hn2d, h_self, w_self, w_neigh, gs, bs, gn, bn):
    n, k = h_self.shape[0], w_self.shape[1]

    def body(pp, hs, ws, wn, gsr, bsr, gnr, bnr, o):
        xs = jnp.dot(hs[...], ws[...], preferred_element_type=jnp.float32)
        zs = _bn_relu(xs, gsr[...], bsr[...])
        hn = pp[...]
        xn = jnp.dot(hn, wn[...], preferred_element_type=jnp.float32)
        zn = _bn_relu(xn, gnr[...], bnr[...])
        z = jnp.maximum(zs + zn, 0.0)
        nrm = jnp.sqrt(jnp.sum(z * z, axis=1, keepdims=True))
        nrm = jnp.where(nrm == 0.0, 1.0, nrm)
        o[...] = z / nrm

    return pl.pallas_call(
        body,
        out_shape=jax.ShapeDtypeStruct((n, k), jnp.float32),
    )(hn2d, h_self, w_self, w_neigh, gs, bs, gn, bn)


def kernel(h_neigh, h_self, edge_index, edge_features, W_edge, W_self, W_neigh,
           gamma_self, beta_self, gamma_neigh, beta_neigh):
    n, d_in = h_neigh.shape
    e = edge_index.shape[1]
    ed = edge_features.shape[1]
    k = W_self.shape[1]

    ch = -(-e // (NW * 128))
    e_pad = NW * ch * 128
    pad = e_pad - e
    src = jnp.concatenate([edge_index[0], jnp.zeros((pad,), jnp.int32)])
    dst = jnp.concatenate([edge_index[1], jnp.zeros((pad,), jnp.int32)])
    ef = jnp.concatenate([edge_features,
                          jnp.zeros((pad, ed), edge_features.dtype)])
    src2d = src.reshape(e_pad // 128, 128)
    dst2d = dst.reshape(e_pad // 128, 128)
    w4 = W_edge.reshape(ed, d_in, k).transpose(1, 0, 2).reshape(d_in, ed * k)
    hsrc = _sc_gather(h_neigh, src2d, e_pad, ch)
    msg = _msg_matmul(hsrc, ef, w4, 2048)
    n_pad = -(-n // 16) * 16
    zeros_nk = jnp.zeros((n_pad // 16, 128), jnp.float32)
    partials = _sc_scatter_add(msg, dst, zeros_nk, n_pad, ch)
    hn_pk = _reduce_partials(partials.reshape(NW, n_pad // 8, 128))
    hn2d = hn_pk.reshape(n_pad, k)[:n]
    return _tail(hn2d, h_self, W_self, W_neigh,
                 gamma_self.reshape(1, k), beta_self.reshape(1, k),
                 gamma_neigh.reshape(1, k), beta_neigh.reshape(1, k))


# speedup vs baseline: 1.1582x; 1.1391x over previous
import dataclasses
import functools

import jax
import jax.numpy as jnp
from jax import lax
from jax.experimental import pallas as pl
from jax.experimental.pallas import tpu as pltpu
from jax.experimental.pallas import tpu_sc as plsc

NC = 2
NS = 16
NW = NC * NS


def _sc_compiler_params():
    cp = pltpu.CompilerParams()
    if "needs_layout_passes" in pltpu.CompilerParams.__dataclass_fields__:
        cp = dataclasses.replace(cp, needs_layout_passes=False)
    return cp


def _sc_gather(table, src2d, e_pad, ch):
    d = table.shape[1]
    mesh = plsc.VectorSubcoreMesh(core_axis_name="c", subcore_axis_name="s")

    @functools.partial(
        pl.kernel,
        mesh=mesh,
        out_type=jax.ShapeDtypeStruct((e_pad, d), table.dtype),
        scratch_types=[
            pltpu.VMEM((ch, 128), jnp.int32),
            pltpu.VMEM((128, d), table.dtype),
            pltpu.VMEM((128, d), table.dtype),
            pltpu.VMEM((128, d), table.dtype),
            pltpu.VMEM((128, d), table.dtype),
            pltpu.SemaphoreType.DMA,
            pltpu.SemaphoreType.DMA,
            pltpu.SemaphoreType.DMA,
            pltpu.SemaphoreType.DMA,
        ],
    )
    def gk(table_hbm, src_hbm, out_hbm, idx_v, b0, b1, b2, b3, s0, s1, s2, s3):
        wid = lax.axis_index("s") * NC + lax.axis_index("c")
        pltpu.sync_copy(src_hbm.at[pl.ds(wid * ch, ch)], idx_v)

        @pl.loop(0, ch, step=4)
        def _(j):
            base = wid * ch * 128 + j * 128
            cps = [pltpu.async_copy(table_hbm.at[idx_v.at[j + q]], b, s)
                   for q, (b, s) in enumerate(((b0, s0), (b1, s1),
                                               (b2, s2), (b3, s3)))]
            for q, (b, cp) in enumerate(zip((b0, b1, b2, b3), cps)):
                cp.wait()
                pltpu.sync_copy(b, out_hbm.at[pl.ds(base + q * 128, 128)])

    return gk(table, src2d)


def _sc_scatter_add(msg, dst1d, zeros_nk, n_pad, ch):
    k = msg.shape[1]
    half = n_pad // 2
    zrows = half // 8
    mesh = plsc.VectorSubcoreMesh(core_axis_name="c", subcore_axis_name="s")

    @functools.partial(
        pl.kernel,
        mesh=mesh,
        out_type=jax.ShapeDtypeStruct((NW, 2, zrows, 128), msg.dtype),
        compiler_params=_sc_compiler_params(),
        scratch_types=[
            pltpu.VMEM((zrows, 128), jnp.float32),
            pltpu.VMEM((128,), jnp.int32),
            pltpu.VMEM((128,), jnp.int32),
            pltpu.VMEM((128, k), jnp.float32),
            pltpu.VMEM((128, k), jnp.float32),
            pltpu.SemaphoreType.DMA,
            pltpu.SemaphoreType.DMA,
        ],
    )
    def sk(msg_hbm, dst_hbm, zeros_hbm, out_hbm, acc, ia, ib, ma, mb, sa, sb):
        c = lax.axis_index("c")
        s = lax.axis_index("s")
        wid = s * NC + c
        col = lax.iota(jnp.int32, 16)
        w0 = wid * ch * 128

        def issue(b, ibuf, mbuf, sem):
            pltpu.async_copy(dst_hbm.at[pl.ds(b, 128)], ibuf, sem)
            pltpu.async_copy(msg_hbm.at[pl.ds(b, 128)], mbuf, sem)

        def drain(b, ibuf, mbuf, sem):
            pltpu.make_async_copy(dst_hbm.at[pl.ds(b, 128)], ibuf, sem).wait()
            pltpu.make_async_copy(msg_hbm.at[pl.ds(b, 128)], mbuf, sem).wait()

        for p in range(2):
            pltpu.sync_copy(zeros_hbm, acc)

            def compute(ibuf, mbuf):
                @pl.loop(0, 8)
                def _(g):
                    rel = ibuf[pl.ds(g * 16, 16)] - p * half
                    rowg = lax.shift_right_arithmetic(rel, 3)
                    cpg = lax.shift_left(rel & 7, 4)
                    mig = ((rel >= 0) & (rel < half)).astype(jnp.int32)
                    for l in range(16):
                        lf = jnp.full((16,), l, jnp.int32)
                        row = rowg.at[lf].get(mode="promise_in_bounds")
                        cp = cpg.at[lf].get(mode="promise_in_bounds") + col
                        m = mig.at[lf].get(mode="promise_in_bounds") != 0
                        vals = mbuf[g * 16 + l, :]
                        plsc.addupdate_scatter(acc, [row, cp], vals, mask=m)

            issue(w0, ia, ma, sa)
            issue(w0 + 128, ib, mb, sb)

            @pl.loop(0, ch, step=2)
            def _(j):
                base = w0 + j * 128
                drain(base, ia, ma, sa)
                compute(ia, ma)

                @pl.when(j + 2 < ch)
                def _():
                    issue(base + 2 * 128, ia, ma, sa)

                drain(base + 128, ib, mb, sb)
                compute(ib, mb)

                @pl.when(j + 3 < ch)
                def _():
                    issue(base + 3 * 128, ib, mb, sb)

            pltpu.sync_copy(acc, out_hbm.at[wid, p])

    return sk(msg, dst1d, zeros_nk)


def _msg_matmul(hsrc, ef, w4, be):
    e_pad, d_in = hsrc.shape
    ed = ef.shape[1]
    k = w4.shape[1] // ed
    dsel = jnp.repeat(jnp.eye(ed, dtype=jnp.float32), k, axis=1)
    rsel = jnp.tile(jnp.eye(k, dtype=jnp.float32), (ed, 1))

    def body(h_ref, ef_ref, w_ref, s_ref, r_ref, o_ref):
        p = jnp.dot(h_ref[...], w_ref[...], preferred_element_type=jnp.float32)
        eft = jnp.dot(ef_ref[...], s_ref[...], preferred_element_type=jnp.float32)
        o_ref[...] = jnp.dot(p * eft, r_ref[...],
                             preferred_element_type=jnp.float32)

    return pl.pallas_call(
        body,
        grid=(e_pad // be,),
        in_specs=[
            pl.BlockSpec((be, d_in), lambda i: (i, 0)),
            pl.BlockSpec((be, ed), lambda i: (i, 0)),
            pl.BlockSpec((d_in, ed * k), lambda i: (0, 0)),
            pl.BlockSpec((ed, ed * k), lambda i: (0, 0)),
            pl.BlockSpec((ed * k, k), lambda i: (0, 0)),
        ],
        out_specs=pl.BlockSpec((be, k), lambda i: (i, 0)),
        out_shape=jax.ShapeDtypeStruct((e_pad, k), jnp.float32),
    )(hsrc, ef, w4, dsel, rsel)


def _reduce_partials(pp):
    nw, r, w = pp.shape

    def body(p_ref, o_ref):
        i = pl.program_id(0)

        @pl.when(i == 0)
        def _():
            o_ref[...] = p_ref[0]

        @pl.when(i > 0)
        def _():
            o_ref[...] += p_ref[0]

    return pl.pallas_call(
        body,
        grid=(nw,),
        in_specs=[pl.BlockSpec((1, r, w), lambda i: (i, 0, 0))],
        out_specs=pl.BlockSpec((r, w), lambda i: (0, 0)),
        out_shape=jax.ShapeDtypeStruct((r, w), jnp.float32),
    )(pp)


def _bn_relu(x, g, b, eps=1e-5):
    mean = jnp.mean(x, axis=0, keepdims=True)
    xc = x - mean
    var = jnp.mean(xc * xc, axis=0, keepdims=True)
    return jnp.maximum(g * xc / jnp.sqrt(var + eps) + b, 0.0)


def _tail(hn2d, h_self, w_self, w_neigh, gs, bs, gn, bn):
    n, k = h_self.shape[0], w_self.shape[1]

    def body(pp, hs, ws, wn, gsr, bsr, gnr, bnr, o):
        xs = jnp.dot(hs[...], ws[...], preferred_element_type=jnp.float32)
        zs = _bn_relu(xs, gsr[...], bsr[...])
        hn = pp[...]
        xn = jnp.dot(hn, wn[...], preferred_element_type=jnp.float32)
        zn = _bn_relu(xn, gnr[...], bnr[...])
        z = jnp.maximum(zs + zn, 0.0)
        nrm = jnp.sqrt(jnp.sum(z * z, axis=1, keepdims=True))
        nrm = jnp.where(nrm == 0.0, 1.0, nrm)
        o[...] = z / nrm

    return pl.pallas_call(
        body,
        out_shape=jax.ShapeDtypeStruct((n, k), jnp.float32),
    )(hn2d, h_self, w_self, w_neigh, gs, bs, gn, bn)


def kernel(h_neigh, h_self, edge_index, edge_features, W_edge, W_self, W_neigh,
           gamma_self, beta_self, gamma_neigh, beta_neigh):
    n, d_in = h_neigh.shape
    e = edge_index.shape[1]
    ed = edge_features.shape[1]
    k = W_self.shape[1]

    ch = -(-e // (NW * 128))
    e_pad = NW * ch * 128
    pad = e_pad - e
    src = jnp.concatenate([edge_index[0], jnp.zeros((pad,), jnp.int32)])
    dst = jnp.concatenate([edge_index[1], jnp.zeros((pad,), jnp.int32)])
    ef = jnp.concatenate([edge_features,
                          jnp.zeros((pad, ed), edge_features.dtype)])
    src2d = src.reshape(e_pad // 128, 128)
    dst2d = dst.reshape(e_pad // 128, 128)
    w4 = W_edge.reshape(ed, d_in, k).transpose(1, 0, 2).reshape(d_in, ed * k)
    hsrc = _sc_gather(h_neigh, src2d, e_pad, ch)
    msg = _msg_matmul(hsrc, ef, w4, 2048)
    n_pad = -(-n // 16) * 16
    zeros_nk = jnp.zeros((n_pad // 16, 128), jnp.float32)
    partials = _sc_scatter_add(msg, dst, zeros_nk, n_pad, ch)
    hn_pk = _reduce_partials(partials.reshape(NW, n_pad // 8, 128))
    hn2d = hn_pk.reshape(n_pad, k)[:n]
    return _tail(hn2d, h_self, W_self, W_neigh,
                 gamma_self.reshape(1, k), beta_self.reshape(1, k),
                 gamma_neigh.reshape(1, k), beta_neigh.reshape(1, k))


# speedup vs baseline: 1.1602x; 1.0017x over previous
import dataclasses
import functools

import jax
import jax.numpy as jnp
from jax import lax
from jax.experimental import pallas as pl
from jax.experimental.pallas import tpu as pltpu
from jax.experimental.pallas import tpu_sc as plsc

NC = 2
NS = 16
NW = NC * NS


def _sc_compiler_params():
    cp = pltpu.CompilerParams()
    if "needs_layout_passes" in pltpu.CompilerParams.__dataclass_fields__:
        cp = dataclasses.replace(cp, needs_layout_passes=False)
    return cp


def _sc_gather(table, src2d, e_pad, ch):
    d = table.shape[1]
    mesh = plsc.VectorSubcoreMesh(core_axis_name="c", subcore_axis_name="s")

    @functools.partial(
        pl.kernel,
        mesh=mesh,
        out_type=jax.ShapeDtypeStruct((e_pad, d), table.dtype),
        scratch_types=[
            pltpu.VMEM((ch, 128), jnp.int32),
            pltpu.VMEM((128, d), table.dtype),
            pltpu.VMEM((128, d), table.dtype),
            pltpu.VMEM((128, d), table.dtype),
            pltpu.VMEM((128, d), table.dtype),
            pltpu.SemaphoreType.DMA,
            pltpu.SemaphoreType.DMA,
            pltpu.SemaphoreType.DMA,
            pltpu.SemaphoreType.DMA,
            pltpu.SemaphoreType.DMA,
        ],
    )
    def gk(table_hbm, src_hbm, out_hbm, idx_v, b0, b1, b2, b3, s0, s1, s2, s3,
           sw):
        wid = lax.axis_index("s") * NC + lax.axis_index("c")
        pltpu.sync_copy(src_hbm.at[pl.ds(wid * ch, ch)], idx_v)
        bufs = (b0, b1, b2, b3)

        def drain_wb(base):
            for q, b in enumerate(bufs):
                pltpu.make_async_copy(
                    b, out_hbm.at[pl.ds(base + q * 128, 128)], sw).wait()

        @pl.loop(0, ch, step=4)
        def _(j):
            base = wid * ch * 128 + j * 128

            @pl.when(j > 0)
            def _():
                drain_wb(base - 4 * 128)

            cps = [pltpu.async_copy(table_hbm.at[idx_v.at[j + q]], b, s)
                   for q, (b, s) in enumerate(((b0, s0), (b1, s1),
                                               (b2, s2), (b3, s3)))]
            for q, (b, cp) in enumerate(zip(bufs, cps)):
                cp.wait()
                pltpu.async_copy(b, out_hbm.at[pl.ds(base + q * 128, 128)], sw)

        drain_wb(wid * ch * 128 + (ch - 4) * 128)

    return gk(table, src2d)


def _sc_scatter_add(msg, dst1d, zeros_nk, n_pad, ch):
    k = msg.shape[1]
    half = n_pad // 2
    zrows = half // 8
    mesh = plsc.VectorSubcoreMesh(core_axis_name="c", subcore_axis_name="s")

    @functools.partial(
        pl.kernel,
        mesh=mesh,
        out_type=jax.ShapeDtypeStruct((NW, 2, zrows, 128), msg.dtype),
        compiler_params=_sc_compiler_params(),
        scratch_types=[
            pltpu.VMEM((zrows, 128), jnp.float32),
            pltpu.VMEM((128,), jnp.int32),
            pltpu.VMEM((128,), jnp.int32),
            pltpu.VMEM((128, k), jnp.float32),
            pltpu.VMEM((128, k), jnp.float32),
            pltpu.SemaphoreType.DMA,
            pltpu.SemaphoreType.DMA,
        ],
    )
    def sk(msg_hbm, dst_hbm, zeros_hbm, out_hbm, acc, ia, ib, ma, mb, sa, sb):
        c = lax.axis_index("c")
        s = lax.axis_index("s")
        wid = s * NC + c
        col = lax.iota(jnp.int32, 16)
        w0 = wid * ch * 128

        def issue(b, ibuf, mbuf, sem):
            pltpu.async_copy(dst_hbm.at[pl.ds(b, 128)], ibuf, sem)
            pltpu.async_copy(msg_hbm.at[pl.ds(b, 128)], mbuf, sem)

        def drain(b, ibuf, mbuf, sem):
            pltpu.make_async_copy(dst_hbm.at[pl.ds(b, 128)], ibuf, sem).wait()
            pltpu.make_async_copy(msg_hbm.at[pl.ds(b, 128)], mbuf, sem).wait()

        for p in range(2):
            pltpu.sync_copy(zeros_hbm, acc)

            def compute(ibuf, mbuf):
                @pl.loop(0, 8)
                def _(g):
                    rel = ibuf[pl.ds(g * 16, 16)] - p * half
                    rowg = lax.shift_right_arithmetic(rel, 3)
                    cpg = lax.shift_left(rel & 7, 4)
                    mig = ((rel >= 0) & (rel < half)).astype(jnp.int32)
                    for l in range(16):
                        lf = jnp.full((16,), l, jnp.int32)
                        row = rowg.at[lf].get(mode="promise_in_bounds")
                        cp = cpg.at[lf].get(mode="promise_in_bounds") + col
                        m = mig.at[lf].get(mode="promise_in_bounds") != 0
                        vals = mbuf[g * 16 + l, :]
                        plsc.addupdate_scatter(acc, [row, cp], vals, mask=m)

            issue(w0, ia, ma, sa)
            issue(w0 + 128, ib, mb, sb)

            @pl.loop(0, ch, step=2)
            def _(j):
                base = w0 + j * 128
                drain(base, ia, ma, sa)
                compute(ia, ma)

                @pl.when(j + 2 < ch)
                def _():
                    issue(base + 2 * 128, ia, ma, sa)

                drain(base + 128, ib, mb, sb)
                compute(ib, mb)

                @pl.when(j + 3 < ch)
                def _():
                    issue(base + 3 * 128, ib, mb, sb)

            pltpu.sync_copy(acc, out_hbm.at[wid, p])

    return sk(msg, dst1d, zeros_nk)


def _msg_matmul(hsrc, ef, w4, be):
    e_pad, d_in = hsrc.shape
    ed = ef.shape[1]
    k = w4.shape[1] // ed
    dsel = jnp.repeat(jnp.eye(ed, dtype=jnp.float32), k, axis=1)
    rsel = jnp.tile(jnp.eye(k, dtype=jnp.float32), (ed, 1))

    def body(h_ref, ef_ref, w_ref, s_ref, r_ref, o_ref):
        p = jnp.dot(h_ref[...], w_ref[...], preferred_element_type=jnp.float32)
        eft = jnp.dot(ef_ref[...], s_ref[...], preferred_element_type=jnp.float32)
        o_ref[...] = jnp.dot(p * eft, r_ref[...],
                             preferred_element_type=jnp.float32)

    return pl.pallas_call(
        body,
        grid=(e_pad // be,),
        in_specs=[
            pl.BlockSpec((be, d_in), lambda i: (i, 0)),
            pl.BlockSpec((be, ed), lambda i: (i, 0)),
            pl.BlockSpec((d_in, ed * k), lambda i: (0, 0)),
            pl.BlockSpec((ed, ed * k), lambda i: (0, 0)),
            pl.BlockSpec((ed * k, k), lambda i: (0, 0)),
        ],
        out_specs=pl.BlockSpec((be, k), lambda i: (i, 0)),
        out_shape=jax.ShapeDtypeStruct((e_pad, k), jnp.float32),
    )(hsrc, ef, w4, dsel, rsel)


def _reduce_partials(pp):
    nw, r, w = pp.shape

    def body(p_ref, o_ref):
        i = pl.program_id(0)

        @pl.when(i == 0)
        def _():
            o_ref[...] = p_ref[0]

        @pl.when(i > 0)
        def _():
            o_ref[...] += p_ref[0]

    return pl.pallas_call(
        body,
        grid=(nw,),
        in_specs=[pl.BlockSpec((1, r, w), lambda i: (i, 0, 0))],
        out_specs=pl.BlockSpec((r, w), lambda i: (0, 0)),
        out_shape=jax.ShapeDtypeStruct((r, w), jnp.float32),
    )(pp)


def _bn_relu(x, g, b, eps=1e-5):
    mean = jnp.mean(x, axis=0, keepdims=True)
    xc = x - mean
    var = jnp.mean(xc * xc, axis=0, keepdims=True)
    return jnp.maximum(g * xc / jnp.sqrt(var + eps) + b, 0.0)


def _tail(hn2d, h_self, w_self, w_neigh, gs, bs, gn, bn):
    n, k = h_self.shape[0], w_self.shape[1]

    def body(pp, hs, ws, wn, gsr, bsr, gnr, bnr, o):
        xs = jnp.dot(hs[...], ws[...], preferred_element_type=jnp.float32)
        zs = _bn_relu(xs, gsr[...], bsr[...])
        hn = pp[...]
        xn = jnp.dot(hn, wn[...], preferred_element_type=jnp.float32)
        zn = _bn_relu(xn, gnr[...], bnr[...])
        z = jnp.maximum(zs + zn, 0.0)
        nrm = jnp.sqrt(jnp.sum(z * z, axis=1, keepdims=True))
        nrm = jnp.where(nrm == 0.0, 1.0, nrm)
        o[...] = z / nrm

    return pl.pallas_call(
        body,
        out_shape=jax.ShapeDtypeStruct((n, k), jnp.float32),
    )(hn2d, h_self, w_self, w_neigh, gs, bs, gn, bn)


def kernel(h_neigh, h_self, edge_index, edge_features, W_edge, W_self, W_neigh,
           gamma_self, beta_self, gamma_neigh, beta_neigh):
    n, d_in = h_neigh.shape
    e = edge_index.shape[1]
    ed = edge_features.shape[1]
    k = W_self.shape[1]

    ch = -(-e // (NW * 128))
    e_pad = NW * ch * 128
    pad = e_pad - e
    src = jnp.concatenate([edge_index[0], jnp.zeros((pad,), jnp.int32)])
    dst = jnp.concatenate([edge_index[1], jnp.zeros((pad,), jnp.int32)])
    ef = jnp.concatenate([edge_features,
                          jnp.zeros((pad, ed), edge_features.dtype)])
    src2d = src.reshape(e_pad // 128, 128)
    dst2d = dst.reshape(e_pad // 128, 128)
    w4 = W_edge.reshape(ed, d_in, k).transpose(1, 0, 2).reshape(d_in, ed * k)
    hsrc = _sc_gather(h_neigh, src2d, e_pad, ch)
    msg = _msg_matmul(hsrc, ef, w4, 2048)
    n_pad = -(-n // 16) * 16
    zeros_nk = jnp.zeros((n_pad // 16, 128), jnp.float32)
    partials = _sc_scatter_add(msg, dst, zeros_nk, n_pad, ch)
    hn_pk = _reduce_partials(partials.reshape(NW, n_pad // 8, 128))
    hn2d = hn_pk.reshape(n_pad, k)[:n]
    return _tail(hn2d, h_self, W_self, W_neigh,
                 gamma_self.reshape(1, k), beta_self.reshape(1, k),
                 gamma_neigh.reshape(1, k), beta_neigh.reshape(1, k))
